# baseline (speedup 1.0000x reference)
import functools

import jax
import jax.numpy as jnp
from jax import lax
from jax.experimental import pallas as pl
from jax.experimental.pallas import tpu as pltpu
from jax.experimental.pallas import tpu_sc as plsc

B = 16384
EMB = 128
NC = 2
NS = 16
NW = NC * NS
CHUNKS = 1
BC = B // CHUNKS
BPW = BC // NW
CH = BPW // 128


NBUF = 4


def _sc_gather(chunk, uid2, iid2, user_emb, item_emb):
    mesh = plsc.VectorSubcoreMesh(core_axis_name="c", subcore_axis_name="s")
    chunk_row = chunk * NW * CH

    @functools.partial(
        pl.kernel,
        mesh=mesh,
        out_type=jax.ShapeDtypeStruct((BC, 2 * EMB), jnp.float32),
        scratch_types=(
            [pltpu.VMEM((CH, 128), jnp.int32),
             pltpu.VMEM((CH, 128), jnp.int32)]
            + [pltpu.VMEM((128, EMB), jnp.float32)] * NBUF
            + [pltpu.SemaphoreType.DMA] * (2 * NBUF)
        ),
    )
    def k(uid_hbm, iid_hbm, uemb_hbm, iemb_hbm, x_out,
          uidx_v, iidx_v, *bufs_sems):
        bufs = bufs_sems[:NBUF]
        gsem = bufs_sems[NBUF:2 * NBUF]
        wsem = bufs_sems[2 * NBUF:]
        wid = lax.axis_index("s") * NC + lax.axis_index("c")
        base = wid * BPW
        pltpu.sync_copy(uid_hbm.at[pl.ds(chunk_row + wid * CH, CH)], uidx_v)
        pltpu.sync_copy(iid_hbm.at[pl.ds(chunk_row + wid * CH, CH)], iidx_v)
        phases = ([(uemb_hbm, uidx_v, j, 0) for j in range(CH)]
                  + [(iemb_hbm, iidx_v, j, EMB) for j in range(CH)])
        P = len(phases)

        def fire(p):
            tbl, idx, j, _ = phases[p]
            return pltpu.async_copy(tbl.at[idx.at[j]], bufs[p % NBUF],
                                    gsem[p % NBUF])

        gh = [None] * P
        wh = [None] * P
        for p in range(min(NBUF, P)):
            gh[p] = fire(p)
        for p in range(P):
            gh[p].wait()
            _, _, j, col = phases[p]
            wh[p] = pltpu.async_copy(
                bufs[p % NBUF],
                x_out.at[pl.ds(base + j * 128, 128), pl.ds(col, EMB)],
                wsem[p % NBUF])
            if p + NBUF < P:
                wh[p].wait()
                gh[p + NBUF] = fire(p + NBUF)
        for p in range(max(0, P - NBUF), P):
            if p + NBUF >= P:
                wh[p].wait()

    return k(uid2, iid2, user_emb, item_emb)


BLK = 2048


def _mlp_body(x_ref, w1_ref, b1_ref, w2_ref, b2_ref,
              w3_ref, b3_ref, wo_ref, bo_ref, out_ref):
    bf = jnp.bfloat16
    zero = jnp.zeros((), dtype=bf)
    h = jnp.maximum(
        (jnp.dot(x_ref[...].astype(bf), w1_ref[...].astype(bf),
                 preferred_element_type=jnp.float32)
         + b1_ref[...]).astype(bf), zero)
    h = jnp.maximum(
        (jnp.dot(h, w2_ref[...].astype(bf),
                 preferred_element_type=jnp.float32)
         + b2_ref[...]).astype(bf), zero)
    h = jnp.maximum(
        (jnp.dot(h, w3_ref[...].astype(bf),
                 preferred_element_type=jnp.float32)
         + b3_ref[...]).astype(bf), zero)
    out_ref[...] = (
        jnp.dot(wo_ref[...].astype(bf), h.T,
                preferred_element_type=jnp.float32) + bo_ref[...])


def _tc_mlp(x, W1, b1, W2, b2, W3, b3, wo_row, bo):
    full = lambda shape: pl.BlockSpec(shape, lambda i: (0, 0))
    return pl.pallas_call(
        _mlp_body,
        grid=(BC // BLK,),
        in_specs=[
            pl.BlockSpec((BLK, 2 * EMB), lambda i: (i, 0)),
            full((256, 256)), full((1, 256)),
            full((256, 128)), full((1, 128)),
            full((128, 64)), full((1, 64)),
            full((1, 64)), full((1, 1)),
        ],
        out_specs=pl.BlockSpec((1, BLK), lambda i: (0, i)),
        out_shape=jax.ShapeDtypeStruct((1, BC), jnp.float32),
    )(x, W1, b1, W2, b2, W3, b3, wo_row, bo)


def kernel(user_id, item_id, user_emb, item_emb, W1, b1, W2, b2, W3, b3,
           Wo, bo):
    uid2 = user_id.astype(jnp.int32).reshape(CHUNKS * NW * CH, 128)
    iid2 = item_id.astype(jnp.int32).reshape(CHUNKS * NW * CH, 128)
    b1r, b2r, b3r = b1.reshape(1, 256), b2.reshape(1, 128), b3.reshape(1, 64)
    bor = bo.reshape(1, 1)
    outs = []
    for c in range(CHUNKS):
        x = _sc_gather(c, uid2, iid2, user_emb, item_emb)
        outs.append(_tc_mlp(x, W1, b1r, W2, b2r, W3, b3r,
                            Wo.reshape(1, 64), bor))
    return jnp.concatenate(outs, axis=1).reshape(B)


# speedup vs baseline: 1.0380x; 1.0380x over previous
import functools

import jax
import jax.numpy as jnp
from jax import lax
from jax.experimental import pallas as pl
from jax.experimental.pallas import tpu as pltpu
from jax.experimental.pallas import tpu_sc as plsc

B = 16384
EMB = 128
NC = 2
NS = 16
NW = NC * NS
CHUNKS = 1
BC = B // CHUNKS
BPW = BC // NW
CH = BPW // 128


NBUF = 4


def _sc_gather(chunk, uid2, iid2, user_emb, item_emb):
    mesh = plsc.VectorSubcoreMesh(core_axis_name="c", subcore_axis_name="s")
    chunk_row = chunk * NW * CH

    @functools.partial(
        pl.kernel,
        mesh=mesh,
        out_type=jax.ShapeDtypeStruct((BC, 2 * EMB), jnp.float32),
        scratch_types=(
            [pltpu.VMEM((CH, 128), jnp.int32),
             pltpu.VMEM((CH, 128), jnp.int32)]
            + [pltpu.VMEM((128, EMB), jnp.float32)] * NBUF
            + [pltpu.SemaphoreType.DMA] * (2 * NBUF)
        ),
    )
    def k(uid_hbm, iid_hbm, uemb_hbm, iemb_hbm, x_out,
          uidx_v, iidx_v, *bufs_sems):
        bufs = bufs_sems[:NBUF]
        gsem = bufs_sems[NBUF:2 * NBUF]
        wsem = bufs_sems[2 * NBUF:]
        wid = lax.axis_index("s") * NC + lax.axis_index("c")
        base = wid * BPW
        pltpu.sync_copy(uid_hbm.at[pl.ds(chunk_row + wid * CH, CH)], uidx_v)
        pltpu.sync_copy(iid_hbm.at[pl.ds(chunk_row + wid * CH, CH)], iidx_v)
        phases = ([(uemb_hbm, uidx_v, j, 0) for j in range(CH)]
                  + [(iemb_hbm, iidx_v, j, EMB) for j in range(CH)])
        P = len(phases)

        def fire(p):
            tbl, idx, j, _ = phases[p]
            return pltpu.async_copy(tbl.at[idx.at[j]], bufs[p % NBUF],
                                    gsem[p % NBUF])

        gh = [None] * P
        wh = [None] * P
        for p in range(min(NBUF, P)):
            gh[p] = fire(p)
        for p in range(P):
            gh[p].wait()
            _, _, j, col = phases[p]
            wh[p] = pltpu.async_copy(
                bufs[p % NBUF],
                x_out.at[pl.ds(base + j * 128, 128), pl.ds(col, EMB)],
                wsem[p % NBUF])
            if p + NBUF < P:
                wh[p].wait()
                gh[p + NBUF] = fire(p + NBUF)
        for p in range(max(0, P - NBUF), P):
            if p + NBUF >= P:
                wh[p].wait()

    return k(uid2, iid2, user_emb, item_emb)


BLK = 4096


def _mlp_body(x_ref, w1_ref, b1_ref, w2_ref, b2_ref,
              w3_ref, b3_ref, wo_ref, bo_ref, out_ref):
    bf = jnp.bfloat16
    zero = jnp.zeros((), dtype=bf)
    h = jnp.maximum(
        (jnp.dot(x_ref[...].astype(bf), w1_ref[...].astype(bf),
                 preferred_element_type=jnp.float32)
         + b1_ref[...]).astype(bf), zero)
    h = jnp.maximum(
        (jnp.dot(h, w2_ref[...].astype(bf),
                 preferred_element_type=jnp.float32)
         + b2_ref[...]).astype(bf), zero)
    h = jnp.maximum(
        (jnp.dot(h, w3_ref[...].astype(bf),
                 preferred_element_type=jnp.float32)
         + b3_ref[...]).astype(bf), zero)
    out_ref[...] = (
        jnp.dot(wo_ref[...].astype(bf), h.T,
                preferred_element_type=jnp.float32) + bo_ref[...])


def _tc_mlp(x, W1, b1, W2, b2, W3, b3, wo_row, bo):
    full = lambda shape: pl.BlockSpec(shape, lambda i: (0, 0))
    return pl.pallas_call(
        _mlp_body,
        grid=(BC // BLK,),
        in_specs=[
            pl.BlockSpec((BLK, 2 * EMB), lambda i: (i, 0)),
            full((256, 256)), full((1, 256)),
            full((256, 128)), full((1, 128)),
            full((128, 64)), full((1, 64)),
            full((1, 64)), full((1, 1)),
        ],
        out_specs=pl.BlockSpec((1, BLK), lambda i: (0, i)),
        out_shape=jax.ShapeDtypeStruct((1, BC), jnp.float32),
    )(x, W1, b1, W2, b2, W3, b3, wo_row, bo)


def kernel(user_id, item_id, user_emb, item_emb, W1, b1, W2, b2, W3, b3,
           Wo, bo):
    uid2 = user_id.astype(jnp.int32).reshape(CHUNKS * NW * CH, 128)
    iid2 = item_id.astype(jnp.int32).reshape(CHUNKS * NW * CH, 128)
    b1r, b2r, b3r = b1.reshape(1, 256), b2.reshape(1, 128), b3.reshape(1, 64)
    bor = bo.reshape(1, 1)
    outs = []
    for c in range(CHUNKS):
        x = _sc_gather(c, uid2, iid2, user_emb, item_emb)
        outs.append(_tc_mlp(x, W1, b1r, W2, b2r, W3, b3r,
                            Wo.reshape(1, 64), bor))
    return jnp.concatenate(outs, axis=1).reshape(B)
